# baseline (speedup 1.0000x reference)
import jax
import jax.numpy as jnp
from jax.experimental import pallas as pl
from jax.experimental.pallas import tpu as pltpu

_K0 = 928981903
_K1 = 3453687069

_B = 32
_V = 1_000_000
_CHUNK = 16384

_ROT_A = (13, 15, 26, 6)
_ROT_B = (17, 29, 16, 24)


def _threefry_bits(n):
    ks0 = jnp.uint32(_K0)
    ks1 = jnp.uint32(_K1)
    ks2 = jnp.uint32(_K0 ^ _K1 ^ 0x1BD11BDA)
    ks = (ks0, ks1, ks2)

    x0 = jnp.full(n.shape, ks0, jnp.uint32)
    x1 = n + ks1
    for i in range(5):
        rots = _ROT_A if i % 2 == 0 else _ROT_B
        for r in rots:
            x0 = x0 + x1
            x1 = (x1 << jnp.uint32(r)) | (x1 >> jnp.uint32(32 - r))
            x1 = x0 ^ x1
        x0 = x0 + ks[(i + 1) % 3]
        x1 = x1 + ks[(i + 2) % 3] + jnp.uint32(i + 1)
    return x0 ^ x1


def _sample_kernel(x_ref, out_ref, best_val, best_idx):
    i = pl.program_id(0)
    c = x_ref.shape[1]

    @pl.when(i == 0)
    def _():
        best_val[...] = jnp.full((_B, 1), -jnp.inf, jnp.float32)
        best_idx[...] = jnp.zeros((_B, 1), jnp.int32)

    row = jax.lax.broadcasted_iota(jnp.uint32, (_B, c), 0)
    col = jax.lax.broadcasted_iota(jnp.uint32, (_B, c), 1)
    base = (i * c).astype(jnp.uint32)
    n = row * jnp.uint32(_V) + col + base

    bits = _threefry_bits(n)
    fb = (bits >> jnp.uint32(9)) | jnp.uint32(0x3F800000)
    f = jax.lax.bitcast_convert_type(fb, jnp.float32) - jnp.float32(1.0)
    u = jnp.maximum(jnp.float32(1e-20), f + jnp.float32(1e-20))
    g = -jnp.log(-jnp.log(u))

    col_global = col.astype(jnp.int32) + i * c
    valid = col_global < _V
    v = jnp.where(valid, x_ref[...] + g, -jnp.inf)

    m = jnp.max(v, axis=1, keepdims=True)
    idx = jnp.min(
        jnp.where(v == m, col_global, jnp.int32(2**30)), axis=1, keepdims=True
    )

    better = m > best_val[...]
    best_val[...] = jnp.where(better, m, best_val[...])
    best_idx[...] = jnp.where(better, idx, best_idx[...])

    @pl.when(i == pl.num_programs(0) - 1)
    def _():
        out_ref[...] = best_idx[...]


@jax.jit
def kernel(logits):
    grid = pl.cdiv(_V, _CHUNK)
    out = pl.pallas_call(
        _sample_kernel,
        grid=(grid,),
        in_specs=[pl.BlockSpec((_B, _CHUNK), lambda i: (0, i))],
        out_specs=pl.BlockSpec((_B, 1), lambda i: (0, 0)),
        out_shape=jax.ShapeDtypeStruct((_B, 1), jnp.int32),
        scratch_shapes=[
            pltpu.VMEM((_B, 1), jnp.float32),
            pltpu.VMEM((_B, 1), jnp.int32),
        ],
    )(logits)
    return out[:, 0].astype(jnp.int64)


# speedup vs baseline: 1.5009x; 1.5009x over previous
import numpy as np
import jax
import jax.numpy as jnp
from jax.experimental import pallas as pl
from jax.experimental.pallas import tpu as pltpu

_K0 = 928981903
_K1 = 3453687069
_K2 = (_K0 ^ _K1 ^ 0x1BD11BDA) & 0xFFFFFFFF

_B = 32
_V = 1_000_000
_CHUNK = 32768
_SUB = 512
_NSUB = _CHUNK // _SUB

_ROT_A = (13, 15, 26, 6)
_ROT_B = (17, 29, 16, 24)

_KS = (_K0, _K1, _K2)
_SCHED = tuple(
    (np.uint32(_KS[(g + 1) % 3]), np.uint32((_KS[(g + 2) % 3] + g + 1) & 0xFFFFFFFF))
    for g in range(5)
)

_A_NP = (
    np.arange(_B, dtype=np.uint64)[:, None] * _V
    + np.arange(_SUB, dtype=np.uint64)[None, :]
    + _K1
) & 0xFFFFFFFF
_A_CONST = _A_NP.astype(np.uint32)
_COL_CONST = np.broadcast_to(
    np.arange(_SUB, dtype=np.int32)[None, :], (_B, _SUB)
).copy()


def _gumbel_from_counter(x1):
    x0 = x1 + jnp.uint32(_K0)
    first = True
    for g in range(5):
        rots = _ROT_A if g % 2 == 0 else _ROT_B
        for r in rots:
            if first:
                first = False
            else:
                x0 = x0 + x1
            t = (x1 << jnp.uint32(r)) | (x1 >> jnp.uint32(32 - r))
            x1 = x0 ^ t
        c0, c1 = _SCHED[g]
        x0 = x0 + c0
        x1 = x1 + c1
    bits = x0 ^ x1
    fb = (bits >> jnp.uint32(9)) | jnp.uint32(0x3F800000)
    f = jax.lax.bitcast_convert_type(fb, jnp.float32) - jnp.float32(1.0)
    u = f + jnp.float32(1e-20)
    return -jnp.log(-jnp.log(u))


def _sample_kernel(x_ref, a_ref, col_ref, out_ref, accv, acci):
    i = pl.program_id(0)

    @pl.when(i == 0)
    def _():
        accv[...] = jnp.full((_B, _SUB), -jnp.inf, jnp.float32)
        acci[...] = jnp.zeros((_B, _SUB), jnp.int32)

    a_const = a_ref[...]
    col_const = col_ref[...]

    def body(j, _):
        b = i * _CHUNK + j * _SUB
        x1 = a_const + b.astype(jnp.uint32)
        g = _gumbel_from_counter(x1)
        v = x_ref[:, pl.ds(j * _SUB, _SUB)] + g
        colg = col_const + b
        upd = (v > accv[...]) & (colg < _V)
        accv[...] = jnp.where(upd, v, accv[...])
        acci[...] = jnp.where(upd, colg, acci[...])
        return 0

    jax.lax.fori_loop(0, _NSUB, body, 0)

    @pl.when(i == pl.num_programs(0) - 1)
    def _():
        av = accv[...]
        m = jnp.max(av, axis=1, keepdims=True)
        idx = jnp.min(
            jnp.where(av == m, acci[...], jnp.int32(2**30)),
            axis=1,
            keepdims=True,
        )
        out_ref[...] = idx


@jax.jit
def kernel(logits):
    grid = pl.cdiv(_V, _CHUNK)
    out = pl.pallas_call(
        _sample_kernel,
        grid=(grid,),
        in_specs=[
            pl.BlockSpec((_B, _CHUNK), lambda i: (0, i)),
            pl.BlockSpec((_B, _SUB), lambda i: (0, 0)),
            pl.BlockSpec((_B, _SUB), lambda i: (0, 0)),
        ],
        out_specs=pl.BlockSpec((_B, 1), lambda i: (0, 0)),
        out_shape=jax.ShapeDtypeStruct((_B, 1), jnp.int32),
        scratch_shapes=[
            pltpu.VMEM((_B, _SUB), jnp.float32),
            pltpu.VMEM((_B, _SUB), jnp.int32),
        ],
    )(logits, jnp.asarray(_A_CONST), jnp.asarray(_COL_CONST))
    return out[:, 0].astype(jnp.int64)


# speedup vs baseline: 1.7612x; 1.1734x over previous
import numpy as np
import jax
import jax.numpy as jnp
from jax.experimental import pallas as pl
from jax.experimental.pallas import tpu as pltpu

_K0 = 928981903
_K1 = 3453687069
_K2 = (_K0 ^ _K1 ^ 0x1BD11BDA) & 0xFFFFFFFF

_B = 32
_V = 1_000_000
_CHUNK = 32768
_SUB = 512
_NSUB = _CHUNK // _SUB
_UNROLL = 32

_ROT_A = (13, 15, 26, 6)
_ROT_B = (17, 29, 16, 24)

_KS = (_K0, _K1, _K2)
_SCHED = tuple(
    (np.uint32(_KS[(g + 1) % 3]), np.uint32((_KS[(g + 2) % 3] + g + 1) & 0xFFFFFFFF))
    for g in range(5)
)

_A_NP = (
    np.arange(_B, dtype=np.uint64)[:, None] * _V
    + np.arange(_SUB, dtype=np.uint64)[None, :]
    + _K1
) & 0xFFFFFFFF
_A_CONST = _A_NP.astype(np.uint32)
_COL_CONST = np.broadcast_to(
    np.arange(_SUB, dtype=np.int32)[None, :], (_B, _SUB)
).copy()


def _gumbel_from_counter(x1):
    x0 = x1 + jnp.uint32(_K0)
    first = True
    for g in range(5):
        rots = _ROT_A if g % 2 == 0 else _ROT_B
        for r in rots:
            if first:
                first = False
            else:
                x0 = x0 + x1
            t = (x1 << jnp.uint32(r)) | (x1 >> jnp.uint32(32 - r))
            x1 = x0 ^ t
        c0, c1 = _SCHED[g]
        x0 = x0 + c0
        x1 = x1 + c1
    bits = x0 ^ x1
    fb = (bits >> jnp.uint32(9)) | jnp.uint32(0x3F800000)
    f = jax.lax.bitcast_convert_type(fb, jnp.float32) - jnp.float32(1.0)
    u = f + jnp.float32(1e-20)
    return -jnp.log(-jnp.log(u))


def _sample_kernel(x_ref, a_ref, col_ref, out_ref, accv, acci):
    i = pl.program_id(0)

    @pl.when(i == 0)
    def _():
        accv[...] = jnp.full((_B, _SUB), -jnp.inf, jnp.float32)
        acci[...] = jnp.zeros((_B, _SUB), jnp.int32)

    a_const = a_ref[...]
    col_const = col_ref[...]

    def body(j, _):
        b = i * _CHUNK + j * _SUB
        x1 = a_const + b.astype(jnp.uint32)
        g = _gumbel_from_counter(x1)
        v = x_ref[:, pl.ds(j * _SUB, _SUB)] + g
        colg = col_const + b
        upd = (v > accv[...]) & (colg < _V)
        accv[...] = jnp.where(upd, v, accv[...])
        acci[...] = jnp.where(upd, colg, acci[...])
        return 0

    jax.lax.fori_loop(0, _NSUB, body, 0, unroll=_UNROLL)

    @pl.when(i == pl.num_programs(0) - 1)
    def _():
        av = accv[...]
        m = jnp.max(av, axis=1, keepdims=True)
        idx = jnp.min(
            jnp.where(av == m, acci[...], jnp.int32(2**30)),
            axis=1,
            keepdims=True,
        )
        out_ref[...] = idx


@jax.jit
def kernel(logits):
    grid = pl.cdiv(_V, _CHUNK)
    out = pl.pallas_call(
        _sample_kernel,
        grid=(grid,),
        in_specs=[
            pl.BlockSpec((_B, _CHUNK), lambda i: (0, i)),
            pl.BlockSpec((_B, _SUB), lambda i: (0, 0)),
            pl.BlockSpec((_B, _SUB), lambda i: (0, 0)),
        ],
        out_specs=pl.BlockSpec((_B, 1), lambda i: (0, 0)),
        out_shape=jax.ShapeDtypeStruct((_B, 1), jnp.int32),
        scratch_shapes=[
            pltpu.VMEM((_B, _SUB), jnp.float32),
            pltpu.VMEM((_B, _SUB), jnp.int32),
        ],
    )(logits, jnp.asarray(_A_CONST), jnp.asarray(_COL_CONST))
    return out[:, 0].astype(jnp.int64)
